# Initial kernel scaffold; baseline (speedup 1.0000x reference)
import jax
import jax.numpy as jnp
from jax.experimental import pallas as pl


def kernel(x, first_factor, last_factor):
    raise NotImplementedError("write your pallas kernel here")


# speedup vs baseline: 9.7675x; 9.7675x over previous
import functools

import jax
import jax.numpy as jnp
from jax.experimental import pallas as pl
from jax.experimental.pallas import tpu as pltpu
from jax.experimental.pallas import tpu_sc as plsc

_GATHER_WINDOW = 128


@functools.partial(jax.jit, static_argnums=(2,))
def _sc_gather(table, idx_2d, num_idx):
    rank = table.shape[1]
    mesh = plsc.VectorSubcoreMesh(core_axis_name="core", subcore_axis_name="subcore")

    @functools.partial(
        pl.kernel,
        out_type=jax.ShapeDtypeStruct((num_idx, rank), table.dtype),
        mesh=mesh,
        compiler_params=pltpu.CompilerParams(use_tc_tiling_on_sc=False),
    )
    def gather_kernel(tbl_hbm, idx_hbm, out_hbm):
        def body(i_vmem, o_vmem):
            pltpu.sync_copy(tbl_hbm.at[i_vmem.at[0]], o_vmem)

        pltpu.emit_pipeline(
            body,
            grid=(num_idx // _GATHER_WINDOW,),
            in_specs=[pl.BlockSpec((1, _GATHER_WINDOW), lambda i: (0, i))],
            out_specs=[pl.BlockSpec((_GATHER_WINDOW, rank), lambda i: (i, 0))],
            core_axis_name=("core", "subcore"),
            dimension_semantics=(pltpu.PARALLEL,),
        )(idx_hbm, out_hbm)

    return gather_kernel(table, idx_2d)


def _mm_body(a_ref, b_ref, o_ref):
    o_ref[...] = jnp.dot(a_ref[...], b_ref[...], preferred_element_type=jnp.float32)


@functools.partial(jax.jit, static_argnums=(2,))
def _tc_project(a, b, block_rows):
    n, k = a.shape
    m = b.shape[1]
    return pl.pallas_call(
        _mm_body,
        grid=(n // block_rows,),
        in_specs=[
            pl.BlockSpec((block_rows, k), lambda i: (i, 0)),
            pl.BlockSpec((k, m), lambda i: (0, 0)),
        ],
        out_specs=pl.BlockSpec((block_rows, m), lambda i: (i, 0)),
        out_shape=jax.ShapeDtypeStruct((n, m), jnp.float32),
    )(a, b)


def kernel(x, first_factor, last_factor):
    emb_dim = last_factor.shape[1]
    x_flat = x.reshape(1, -1).astype(jnp.int32)
    num_idx = x_flat.shape[1]
    gathered = _sc_gather(first_factor, x_flat, num_idx)
    out = _tc_project(gathered, last_factor, 2048)
    return out.reshape(tuple(x.shape) + (emb_dim,))


# speedup vs baseline: 11.2221x; 1.1489x over previous
import functools

import jax
import jax.numpy as jnp
from jax.experimental import pallas as pl
from jax.experimental.pallas import tpu as pltpu
from jax.experimental.pallas import tpu_sc as plsc

_W = 128
_PACK_BLOCK = 8192
_MM_BLOCK = 512


def _pack_body(a_ref, o_ref):
    a = a_ref[...]
    a3 = a.reshape(a.shape[0] // 4, 4, 32)
    for j in range(4):
        o_ref[:, 32 * j:32 * (j + 1)] = a3[:, j, :]


@jax.jit
def _tc_pack(table):
    n, r = table.shape
    return pl.pallas_call(
        _pack_body,
        grid=(n // _PACK_BLOCK,),
        in_specs=[pl.BlockSpec((_PACK_BLOCK, r), lambda i: (i, 0))],
        out_specs=pl.BlockSpec((_PACK_BLOCK // 4, 4 * r), lambda i: (i, 0)),
        out_shape=jax.ShapeDtypeStruct((n // 4, 4 * r), table.dtype),
    )(table)


@functools.partial(jax.jit, static_argnums=(2, 3))
def _sc_gather(packed_table, idx_2d, num_rows, rank):
    n_steps = idx_2d.shape[0]
    mesh = plsc.VectorSubcoreMesh(core_axis_name="core", subcore_axis_name="subcore")

    @functools.partial(
        pl.kernel,
        out_type=jax.ShapeDtypeStruct((n_steps * _W, rank), jnp.float32),
        mesh=mesh,
        compiler_params=pltpu.CompilerParams(use_tc_tiling_on_sc=False),
    )
    def gather_kernel(tbl_hbm, idx_hbm, out_hbm):
        def body(i_vmem, o_vmem):
            pltpu.sync_copy(tbl_hbm.at[i_vmem.at[0]], o_vmem)

        pltpu.emit_pipeline(
            body,
            grid=(n_steps,),
            in_specs=[pl.BlockSpec((1, _W), lambda i: (i, 0))],
            out_specs=[pl.BlockSpec((_W, rank), lambda i: (i, 0))],
            core_axis_name=("core", "subcore"),
            dimension_semantics=(pltpu.PARALLEL,),
        )(idx_hbm, out_hbm)

    return gather_kernel(packed_table, idx_2d)


def _mm_body(a_ref, b_ref, o_ref):
    a = a_ref[...].astype(jnp.bfloat16)
    o_ref[...] = jnp.dot(a, b_ref[...], preferred_element_type=jnp.float32)


@jax.jit
def _tc_project(a_packed, lb):
    n = a_packed.shape[0]
    m = lb.shape[1]
    return pl.pallas_call(
        _mm_body,
        grid=(n // _MM_BLOCK,),
        in_specs=[
            pl.BlockSpec((_MM_BLOCK, 128), lambda i: (i, 0)),
            pl.BlockSpec((128, m), lambda i: (0, 0)),
        ],
        out_specs=pl.BlockSpec((_MM_BLOCK, m), lambda i: (i, 0)),
        out_shape=jax.ShapeDtypeStruct((n, m), jnp.float32),
    )(a_packed, lb)


def kernel(x, first_factor, last_factor):
    num_rows, rank = first_factor.shape
    emb_dim = last_factor.shape[1]
    num_idx = x.size

    idx_2d = x.reshape(-1).astype(jnp.int32).reshape(num_idx // _W, _W)
    packed_table = _tc_pack(first_factor).reshape(num_rows, rank)
    gathered = _sc_gather(packed_table, idx_2d, num_rows, rank)
    gathered_packed = gathered.reshape(num_idx * rank // 128, 128)
    lb = jnp.kron(jnp.eye(4, dtype=jnp.float32), last_factor).astype(jnp.bfloat16)
    out = _tc_project(gathered_packed, lb)
    return out.reshape(tuple(x.shape) + (emb_dim,))
